# baseline (speedup 1.0000x reference)
import functools

import jax
import jax.numpy as jnp
from jax.experimental import pallas as pl
from jax.experimental.pallas import tpu as pltpu
from jax.experimental.pallas import tpu_sc as plsc

VOCAB = 20000
EMB = 50
SEQ = 42
BATCH = 16384
HID = 128
DPAD = 128
GATHER_WINDOW = 128
BLOCK_B = 512


def _sc_gather(table_pad, idx2d):
    n = idx2d.shape[1]
    mesh = plsc.VectorSubcoreMesh(core_axis_name="core", subcore_axis_name="subcore")

    @functools.partial(
        pl.kernel,
        out_type=jax.ShapeDtypeStruct((n, DPAD), table_pad.dtype),
        mesh=mesh,
    )
    def gather_kernel(table_hbm, i_hbm, o_hbm):
        def body(i_vmem, o_vmem):
            pltpu.sync_copy(table_hbm.at[i_vmem.at[0]], o_vmem)

        pltpu.emit_pipeline(
            body,
            grid=(n // GATHER_WINDOW,),
            in_specs=[pl.BlockSpec((1, GATHER_WINDOW), lambda i: (0, i))],
            out_specs=[pl.BlockSpec((GATHER_WINDOW, DPAD), lambda i: (i, 0))],
            core_axis_name=("core", "subcore"),
            dimension_semantics=(pltpu.PARALLEL,),
        )(i_hbm, o_hbm)

    return gather_kernel(table_pad, idx2d)


def _mlp_body(x_ref, w1_ref, b1_ref, w2_ref, b2_ref, o_ref, acc_ref):
    s = pl.program_id(1)

    @pl.when(s == 0)
    def _():
        acc_ref[...] = jnp.zeros_like(acc_ref)

    acc_ref[...] += jnp.dot(
        x_ref[:, 0, 0, :], w1_ref[0], preferred_element_type=jnp.float32
    )

    @pl.when(s == SEQ - 1)
    def _():
        h = jnp.maximum(acc_ref[...] + b1_ref[...], 0.0)
        o = jnp.dot(h, w2_ref[...], preferred_element_type=jnp.float32) + b2_ref[...]
        o_ref[...] = jax.nn.sigmoid(o)


def _tc_mlp(x4, w1r, b1, w2, b2):
    grid = (BATCH // BLOCK_B, SEQ)
    return pl.pallas_call(
        _mlp_body,
        grid=grid,
        in_specs=[
            pl.BlockSpec((BLOCK_B, 1, 1, DPAD), lambda i, s: (i, s, 0, 0)),
            pl.BlockSpec((1, DPAD, HID), lambda i, s: (s, 0, 0)),
            pl.BlockSpec((1, HID), lambda i, s: (0, 0)),
            pl.BlockSpec((HID, 1), lambda i, s: (0, 0)),
            pl.BlockSpec((1, 1), lambda i, s: (0, 0)),
        ],
        out_specs=pl.BlockSpec((BLOCK_B, 1), lambda i, s: (i, 0)),
        out_shape=jax.ShapeDtypeStruct((BATCH, 1), jnp.float32),
        scratch_shapes=[pltpu.VMEM((BLOCK_B, HID), jnp.float32)],
    )(x4, w1r, b1.reshape(1, HID), w2, b2.reshape(1, 1))


def kernel(indices, table, W1, b1, W2, b2):
    table_pad = jnp.pad(table, ((0, 0), (0, DPAD - EMB)))
    idx2d = indices.astype(jnp.int32).reshape(1, BATCH * SEQ)
    x = _sc_gather(table_pad, idx2d)
    x4 = x.reshape(BATCH, SEQ, 1, DPAD)
    w1r = jnp.pad(W1.reshape(SEQ, EMB, HID), ((0, 0), (0, DPAD - EMB), (0, 0)))
    return _tc_mlp(x4, w1r, b1, W2, b2)
